# baseline (speedup 1.0000x reference)
import functools

import jax
import jax.numpy as jnp
from jax import lax
from jax.experimental import pallas as pl
from jax.experimental.pallas import tpu as pltpu
from jax.experimental.pallas import tpu_sc as plsc


CHUNK = 128


def _dot(a, b):
    return jnp.dot(a, b, preferred_element_type=jnp.float32)


def _entry_body(x_ref, w_in_ref, b_in_ref, wm_ref, bm_ref, state_ref, msg_ref):
    state = jax.nn.relu(_dot(x_ref[...], w_in_ref[...]) + b_in_ref[...])
    state_ref[...] = state
    msg_ref[...] = jax.nn.relu(_dot(state, wm_ref[...]) + bm_ref[...])


def _round_body(p0_ref, p1_ref, state_ref, wu_ref, bu_ref, wm_ref, bm_ref,
                new_state_ref, msg_ref):
    agg = p0_ref[...] + p1_ref[...]
    state = state_ref[...] + jax.nn.relu(_dot(agg, wu_ref[...]) + bu_ref[...])
    new_state_ref[...] = state
    msg_ref[...] = jax.nn.relu(_dot(state, wm_ref[...]) + bm_ref[...])


def _final_body(p0_ref, p1_ref, state_ref, wu_ref, bu_ref,
                wmu_ref, bmu_ref, wlv_ref, blv_ref, mu_ref, lv_ref):
    agg = p0_ref[...] + p1_ref[...]
    state = state_ref[...] + jax.nn.relu(_dot(agg, wu_ref[...]) + bu_ref[...])
    mu_ref[...] = _dot(state, wmu_ref[...]) + bmu_ref[...]
    lv_ref[...] = _dot(state, wlv_ref[...]) + blv_ref[...]


def _row_block(n_rows, d, block_rows):
    return pl.BlockSpec((block_rows, d), lambda i: (i, 0))


def _full_block(shape):
    return pl.BlockSpec(shape, lambda i: tuple(0 for _ in shape))


def _make_tc_call(body, n, block_rows, in_shapes, out_shapes):
    def spec(s):
        if s[0] == "rows":
            return _row_block(n, s[1], block_rows)
        return _full_block(s)

    grid = (n // block_rows,)
    return pl.pallas_call(
        body,
        grid=grid,
        in_specs=[spec(s) for s in in_shapes],
        out_specs=[spec(s) for s in out_shapes],
        out_shape=[
            jax.ShapeDtypeStruct((n, s[1]) if s[0] == "rows" else s,
                                 jnp.float32)
            for s in out_shapes
        ],
    )


def _make_sc_scatter(n_pad, n_chunks, s_dim):
    cpt = n_chunks // 32
    rows_per_tile = n_pad // 16
    mesh = plsc.VectorSubcoreMesh(core_axis_name="c", subcore_axis_name="s")

    @functools.partial(
        pl.kernel,
        mesh=mesh,
        out_type=jax.ShapeDtypeStruct((2, n_pad, s_dim), jnp.float32),
        scratch_types=[
            pltpu.VMEM((CHUNK,), jnp.int32),
            pltpu.VMEM((CHUNK,), jnp.int32),
            pltpu.VMEM((CHUNK, s_dim), jnp.float32),
            pltpu.VMEM_SHARED((n_pad, s_dim), jnp.float32),
            pltpu.SemaphoreType.DMA,
        ],
    )
    def sc_kernel(msg_hbm, src_hbm, dst_hbm, zeros_hbm, out_hbm,
                  idx_s, idx_d, rows, agg_sh, sem):
        cid = lax.axis_index("c")
        sid = lax.axis_index("s")
        wid = sid * 2 + cid

        row0 = sid * rows_per_tile
        pltpu.sync_copy(zeros_hbm.at[pl.ds(row0, rows_per_tile)],
                        agg_sh.at[pl.ds(row0, rows_per_tile)])
        plsc.subcore_barrier()

        def body(j, carry):
            base = (wid + j * 32) * CHUNK
            pltpu.sync_copy(src_hbm.at[pl.ds(base, CHUNK)], idx_s)
            pltpu.sync_copy(dst_hbm.at[pl.ds(base, CHUNK)], idx_d)
            pltpu.async_copy(msg_hbm.at[idx_s], rows, sem).wait()
            pltpu.sync_copy(rows, agg_sh.at[idx_d], add=True)
            return carry

        lax.fori_loop(0, cpt, body, 0)
        plsc.subcore_barrier()

        pltpu.sync_copy(agg_sh.at[pl.ds(row0, rows_per_tile)],
                        out_hbm.at[cid, pl.ds(row0, rows_per_tile)])

    return sc_kernel


def kernel(x, edge_index, W_in, b_in, Wm, bm, Wu, bu, W_mu, b_mu, W_lv, b_lv):
    n, d = x.shape
    s_dim = W_in.shape[1]
    l_dim = W_mu.shape[1]
    r_rounds = Wm.shape[0]
    e = edge_index.shape[1]

    src = edge_index[0].astype(jnp.int32)
    dst = edge_index[1].astype(jnp.int32)

    n_pad = -(-n // 128) * 128

    n_chunks = -(-e // (64 * CHUNK)) * 64
    e_pad = n_chunks * CHUNK
    src = jnp.pad(src, (0, e_pad - e))
    pad_dst = n + jnp.arange(e_pad - e, dtype=jnp.int32) % (n_pad - n)
    dst = jnp.concatenate([dst, pad_dst])
    zeros = jnp.zeros((n_pad, s_dim), jnp.float32)

    block_rows = 1000
    assert n % block_rows == 0 and n < n_pad

    b_in2 = b_in.reshape(1, -1)
    bm2 = bm.reshape(r_rounds, 1, -1)
    bu2 = bu.reshape(r_rounds, 1, -1)
    b_mu2 = b_mu.reshape(1, -1)
    b_lv2 = b_lv.reshape(1, -1)

    rows = ("rows", s_dim)
    wss = (s_dim, s_dim)
    bs = (1, s_dim)

    entry = _make_tc_call(
        _entry_body, n, block_rows,
        [("rows", d), (d, s_dim), bs, wss, bs], [rows, rows])
    round_call = _make_tc_call(
        _round_body, n, block_rows,
        [rows, rows, rows, wss, bs, wss, bs], [rows, rows])
    final_call = _make_tc_call(
        _final_body, n, block_rows,
        [rows, rows, rows, wss, bs, (s_dim, l_dim), (1, l_dim),
         (s_dim, l_dim), (1, l_dim)],
        [("rows", l_dim), ("rows", l_dim)])
    sc_scatter = _make_sc_scatter(n_pad, n_chunks, s_dim)

    state, msg = entry(x, W_in, b_in2, Wm[0], bm2[0])
    for r in range(r_rounds):
        parts = sc_scatter(msg, src, dst, zeros)
        if r + 1 < r_rounds:
            state, msg = round_call(parts[0], parts[1], state,
                                    Wu[r], bu2[r], Wm[r + 1], bm2[r + 1])
        else:
            mu, logvar = final_call(parts[0], parts[1], state,
                                    Wu[r], bu2[r], W_mu, b_mu2, W_lv, b_lv2)
    return (mu, logvar)


# speedup vs baseline: 1.9746x; 1.9746x over previous
import functools

import jax
import jax.numpy as jnp
from jax import lax
from jax.experimental import pallas as pl
from jax.experimental.pallas import tpu as pltpu
from jax.experimental.pallas import tpu_sc as plsc


CHUNK = 128


def _dot(a, b):
    return jnp.dot(a, b, preferred_element_type=jnp.float32)


def _entry_body(x_ref, w_in_ref, b_in_ref, wm_ref, bm_ref, state_ref, msg_ref):
    state = jax.nn.relu(_dot(x_ref[...], w_in_ref[...]) + b_in_ref[...])
    state_ref[...] = state
    msg_ref[...] = jax.nn.relu(_dot(state, wm_ref[...]) + bm_ref[...])


def _round_body(p0_ref, p1_ref, state_ref, wu_ref, bu_ref, wm_ref, bm_ref,
                new_state_ref, msg_ref):
    agg = p0_ref[...] + p1_ref[...]
    state = state_ref[...] + jax.nn.relu(_dot(agg, wu_ref[...]) + bu_ref[...])
    new_state_ref[...] = state
    msg_ref[...] = jax.nn.relu(_dot(state, wm_ref[...]) + bm_ref[...])


def _final_body(p0_ref, p1_ref, state_ref, wu_ref, bu_ref,
                wmu_ref, bmu_ref, wlv_ref, blv_ref, mu_ref, lv_ref):
    agg = p0_ref[...] + p1_ref[...]
    state = state_ref[...] + jax.nn.relu(_dot(agg, wu_ref[...]) + bu_ref[...])
    mu_ref[...] = _dot(state, wmu_ref[...]) + bmu_ref[...]
    lv_ref[...] = _dot(state, wlv_ref[...]) + blv_ref[...]


def _row_block(n_rows, d, block_rows):
    return pl.BlockSpec((block_rows, d), lambda i: (i, 0))


def _full_block(shape):
    return pl.BlockSpec(shape, lambda i: tuple(0 for _ in shape))


def _make_tc_call(body, n, block_rows, in_shapes, out_shapes):
    def spec(s):
        if s[0] == "rows":
            return _row_block(n, s[1], block_rows)
        return _full_block(s)

    grid = (n // block_rows,)
    return pl.pallas_call(
        body,
        grid=grid,
        in_specs=[spec(s) for s in in_shapes],
        out_specs=[spec(s) for s in out_shapes],
        out_shape=[
            jax.ShapeDtypeStruct((n, s[1]) if s[0] == "rows" else s,
                                 jnp.float32)
            for s in out_shapes
        ],
    )


def _make_sc_scatter(n_pad, n_chunks, s_dim):
    cpt = -(-n_chunks // 32)
    rows_per_tile = n_pad // 16
    mesh = plsc.VectorSubcoreMesh(core_axis_name="c", subcore_axis_name="s")

    @functools.partial(
        pl.kernel,
        mesh=mesh,
        out_type=jax.ShapeDtypeStruct((2, n_pad, s_dim), jnp.float32),
        scratch_types=[
            pltpu.VMEM((CHUNK,), jnp.int32),
            pltpu.VMEM((CHUNK,), jnp.int32),
            pltpu.VMEM((CHUNK, s_dim), jnp.float32),
            pltpu.VMEM_SHARED((n_pad, s_dim), jnp.float32),
            pltpu.SemaphoreType.DMA,
        ],
    )
    def sc_kernel(msg_hbm, src_hbm, dst_hbm, zeros_hbm, out_hbm,
                  idx_s, idx_d, rows, agg_sh, sem):
        cid = lax.axis_index("c")
        sid = lax.axis_index("s")
        wid = sid * 2 + cid

        row0 = sid * rows_per_tile
        pltpu.sync_copy(zeros_hbm.at[pl.ds(row0, rows_per_tile)],
                        agg_sh.at[pl.ds(row0, rows_per_tile)])
        plsc.subcore_barrier()

        def body(j, carry):
            chunk = wid + j * 32

            @pl.when(chunk < n_chunks)
            def _():
                base = chunk * CHUNK
                pltpu.sync_copy(src_hbm.at[pl.ds(base, CHUNK)], idx_s)
                pltpu.sync_copy(dst_hbm.at[pl.ds(base, CHUNK)], idx_d)
                pltpu.async_copy(msg_hbm.at[idx_s], rows, sem).wait()
                pltpu.sync_copy(rows, agg_sh.at[idx_d], add=True)

            return carry

        lax.fori_loop(0, cpt, body, 0)
        plsc.subcore_barrier()

        pltpu.sync_copy(agg_sh.at[pl.ds(row0, rows_per_tile)],
                        out_hbm.at[cid, pl.ds(row0, rows_per_tile)])

    return sc_kernel


def kernel(x, edge_index, W_in, b_in, Wm, bm, Wu, bu, W_mu, b_mu, W_lv, b_lv):
    n, d = x.shape
    s_dim = W_in.shape[1]
    l_dim = W_mu.shape[1]
    r_rounds = Wm.shape[0]
    e = edge_index.shape[1]

    src = edge_index[0].astype(jnp.int32)
    dst = edge_index[1].astype(jnp.int32)

    n_pad = -(-n // 128) * 128

    n_chunks = -(-e // CHUNK)
    e_pad = n_chunks * CHUNK
    src = jnp.pad(src, (0, e_pad - e))
    pad_dst = n + jnp.arange(e_pad - e, dtype=jnp.int32) % (n_pad - n)
    dst = jnp.concatenate([dst, pad_dst])
    zeros = jnp.zeros((n_pad, s_dim), jnp.float32)

    block_rows = 1000
    assert n % block_rows == 0 and n < n_pad

    b_in2 = b_in.reshape(1, -1)
    bm2 = bm.reshape(r_rounds, 1, -1)
    bu2 = bu.reshape(r_rounds, 1, -1)
    b_mu2 = b_mu.reshape(1, -1)
    b_lv2 = b_lv.reshape(1, -1)

    rows = ("rows", s_dim)
    wss = (s_dim, s_dim)
    bs = (1, s_dim)

    entry = _make_tc_call(
        _entry_body, n, block_rows,
        [("rows", d), (d, s_dim), bs, wss, bs], [rows, rows])
    round_call = _make_tc_call(
        _round_body, n, block_rows,
        [rows, rows, rows, wss, bs, wss, bs], [rows, rows])
    final_call = _make_tc_call(
        _final_body, n, block_rows,
        [rows, rows, rows, wss, bs, (s_dim, l_dim), (1, l_dim),
         (s_dim, l_dim), (1, l_dim)],
        [("rows", l_dim), ("rows", l_dim)])
    sc_scatter = _make_sc_scatter(n_pad, n_chunks, s_dim)

    state, msg = entry(x, W_in, b_in2, Wm[0], bm2[0])
    for r in range(r_rounds):
        parts = sc_scatter(msg, src, dst, zeros)
        if r + 1 < r_rounds:
            state, msg = round_call(parts[0], parts[1], state,
                                    Wu[r], bu2[r], Wm[r + 1], bm2[r + 1])
        else:
            mu, logvar = final_call(parts[0], parts[1], state,
                                    Wu[r], bu2[r], W_mu, b_mu2, W_lv, b_lv2)
    return (mu, logvar)
